# baseline (speedup 1.0000x reference)
import functools

import jax
import jax.numpy as jnp
from jax import lax
from jax.experimental import pallas as pl
from jax.experimental.pallas import tpu as pltpu
from jax.experimental.pallas import tpu_sc as plsc

N_EDGES = 500000
NC, NS, L = 2, 16, 16
NW = NC * NS
C = 128
CPW0 = 152
CPW1 = 96
CPWMAX = max(CPW0, CPW1)
CPWMIN = min(CPW0, CPW1)
N_PAD = NS * (CPW0 + CPW1) * C
D = 128
NBUF = 4
EPS2 = 1e-16


def _rsqrt_nr(x):
    i = plsc.bitcast(x, jnp.int32)
    y = plsc.bitcast(jnp.int32(0x5F3759DF) - (i >> 1), jnp.float32)
    for _ in range(3):
        y = y * (1.5 - 0.5 * x * y * y)
    return y


def _sc_cosine(x_user, x_job, idx_s, idx_d):
    mesh = plsc.VectorSubcoreMesh(core_axis_name="c", subcore_axis_name="s")

    @functools.partial(
        pl.kernel,
        mesh=mesh,
        compiler_params=pltpu.CompilerParams(
            needs_layout_passes=False, use_tc_tiling_on_sc=False),
        out_type=jax.ShapeDtypeStruct((N_PAD,), jnp.float32),
        scratch_types=[
            pltpu.VMEM((CPWMAX * C,), jnp.int32),
            pltpu.VMEM((CPWMAX * C,), jnp.int32),
            pltpu.VMEM((NBUF, C, D // 2), jnp.int32),
            pltpu.VMEM((NBUF, C, D // 2), jnp.int32),
            pltpu.VMEM((CPWMAX * C,), jnp.float32),
            pltpu.SemaphoreType.DMA,
            pltpu.SemaphoreType.DMA,
            pltpu.SemaphoreType.DMA,
            pltpu.SemaphoreType.DMA,
            pltpu.SemaphoreType.DMA,
        ],
    )
    def k(xu_hbm, xj_hbm, is_hbm, id_hbm, out_hbm,
          idx_sv, idx_dv, rows_s, rows_d, out_v, g0, g1, g2, g3, si):
        cid = lax.axis_index("c")
        sid = lax.axis_index("s")
        wbase = jnp.where(cid == 0, sid * (CPW0 * C),
                          NS * (CPW0 * C) + sid * (CPW1 * C))
        cpw = jnp.where(cid == 0, CPW0, CPW1)
        lane = lax.iota(jnp.int32, L)
        gsem = (g0, g1, g2, g3)

        ci1 = pltpu.async_copy(is_hbm.at[pl.ds(wbase, CPWMIN * C)],
                               idx_sv.at[pl.ds(0, CPWMIN * C)], si)
        ci2 = pltpu.async_copy(id_hbm.at[pl.ds(wbase, CPWMIN * C)],
                               idx_dv.at[pl.ds(0, CPWMIN * C)], si)
        ci1.wait()
        ci2.wait()

        @pl.when(cpw > CPWMIN)
        def _idx_tail():
            rem = (CPWMAX - CPWMIN) * C
            c1 = pltpu.async_copy(is_hbm.at[pl.ds(wbase + CPWMIN * C, rem)],
                                  idx_sv.at[pl.ds(CPWMIN * C, rem)], si)
            c2 = pltpu.async_copy(id_hbm.at[pl.ds(wbase + CPWMIN * C, rem)],
                                  idx_dv.at[pl.ds(CPWMIN * C, rem)], si)
            c1.wait()
            c2.wait()

        def issue(c, b):
            pltpu.async_copy(
                xu_hbm.at[idx_sv.at[pl.ds(c * C, C)]], rows_s.at[b], gsem[b])
            pltpu.async_copy(
                xj_hbm.at[idx_dv.at[pl.ds(c * C, C)]], rows_d.at[b], gsem[b])

        def wait(b):
            pltpu.make_async_copy(
                xu_hbm.at[idx_sv.at[pl.ds(0, C)]], rows_s.at[b], gsem[b]).wait()
            pltpu.make_async_copy(
                xj_hbm.at[idx_dv.at[pl.ds(0, C)]], rows_d.at[b], gsem[b]).wait()

        fifteen = jnp.full((L,), L - 1, jnp.int32)
        unpack = functools.partial(
            plsc.unpack, format=plsc.PackFormat.INTERLEAVED)

        def compute(c, b):
            rs_ref = rows_s.at[b]
            rd_ref = rows_d.at[b]

            def group_body(g, _):
                gbase = g * L
                z = jnp.zeros((L,), jnp.float32)
                dotv, nsv, ndv = z, z, z
                for e in range(L):
                    row = gbase + e
                    acc_d, acc_s, acc_t = z, z, z
                    for j in range(D // (2 * L)):
                        xs = plsc.bitcast(
                            rs_ref[row, pl.ds(j * L, L)], jnp.bfloat16)
                        xd = plsc.bitcast(
                            rd_ref[row, pl.ds(j * L, L)], jnp.bfloat16)
                        sa, sb = unpack(xs)
                        da, db = unpack(xd)
                        acc_d = acc_d + sa * da + sb * db
                        acc_s = acc_s + sa * sa + sb * sb
                        acc_t = acc_t + da * da + db * db
                    td = plsc.cumsum(acc_d).at[fifteen].get(
                        mode="promise_in_bounds")
                    ts = plsc.cumsum(acc_s).at[fifteen].get(
                        mode="promise_in_bounds")
                    tt = plsc.cumsum(acc_t).at[fifteen].get(
                        mode="promise_in_bounds")
                    m = lane == e
                    dotv = jnp.where(m, td, dotv)
                    nsv = jnp.where(m, ts, nsv)
                    ndv = jnp.where(m, tt, ndv)
                rs = _rsqrt_nr(jnp.maximum(nsv, EPS2))
                rd = _rsqrt_nr(jnp.maximum(ndv, EPS2))
                out_v[pl.ds(c * C + gbase, L)] = dotv * rs * rd
                return _

            lax.fori_loop(0, C // L, group_body, None)

        for b in range(NBUF - 1):
            issue(b, b)

        def quad_body(cc, _):
            for b in range(NBUF):
                c = cc * NBUF + b
                nxt = c + NBUF - 1

                @pl.when(nxt < cpw)
                def _prefetch():
                    issue(nxt, (b + NBUF - 1) % NBUF)

                wait(b)
                compute(c, b)
            return _

        lax.fori_loop(0, cpw // NBUF, quad_body, None)
        pltpu.sync_copy(out_v.at[pl.ds(0, CPWMIN * C)],
                        out_hbm.at[pl.ds(wbase, CPWMIN * C)])

        @pl.when(cpw > CPWMIN)
        def _out_tail():
            rem = (CPWMAX - CPWMIN) * C
            pltpu.sync_copy(
                out_v.at[pl.ds(CPWMIN * C, rem)],
                out_hbm.at[pl.ds(wbase + CPWMIN * C, rem)])

    return k(x_user, x_job, idx_s, idx_d)


def _pack_body(x_ref, o_ref):
    u = pltpu.bitcast(x_ref[...].astype(jnp.bfloat16), jnp.uint16)
    lo = u[:, :D // 2].astype(jnp.uint32)
    hi = u[:, D // 2:].astype(jnp.uint32)
    o_ref[...] = pltpu.bitcast(lo | (hi << 16), jnp.int32)


def _tc_pack(x):
    rows = x.shape[0]
    br = 400
    return pl.pallas_call(
        _pack_body,
        out_shape=jax.ShapeDtypeStruct((rows, D // 2), jnp.int32),
        grid=(rows // br,),
        in_specs=[pl.BlockSpec((br, D), lambda i: (i, 0))],
        out_specs=pl.BlockSpec((br, D // 2), lambda i: (i, 0)),
    )(x)


def kernel(x_user, x_job, edge_label_index):
    idx = edge_label_index.astype(jnp.int32)
    pad = N_PAD - N_EDGES
    idx_s = jnp.pad(idx[0], (0, pad))
    idx_d = jnp.pad(idx[1], (0, pad))
    out = _sc_cosine(_tc_pack(x_user), _tc_pack(x_job), idx_s, idx_d)
    return out[:N_EDGES]


# speedup vs baseline: 1.2457x; 1.2457x over previous
import functools

import jax
import jax.numpy as jnp
from jax import lax
from jax.experimental import pallas as pl
from jax.experimental.pallas import tpu as pltpu
from jax.experimental.pallas import tpu_sc as plsc

N_EDGES = 500000
NC, NS, L = 2, 16, 16
NW = NC * NS
C = 128
CPW0 = 152
CPW1 = 96
CPWMAX = max(CPW0, CPW1)
CPWMIN = min(CPW0, CPW1)
N_PAD = NS * (CPW0 + CPW1) * C
D = 128
NBUF = 4
EPS2 = 1e-16


def _rsqrt_nr(x):
    i = plsc.bitcast(x, jnp.int32)
    y = plsc.bitcast(jnp.int32(0x5F3759DF) - (i >> 1), jnp.float32)
    for _ in range(3):
        y = y * (1.5 - 0.5 * x * y * y)
    return y


def _sc_cosine(x_user, x_job, idx_s, idx_d):
    mesh = plsc.VectorSubcoreMesh(core_axis_name="c", subcore_axis_name="s")

    @functools.partial(
        pl.kernel,
        mesh=mesh,
        compiler_params=pltpu.CompilerParams(
            needs_layout_passes=False, use_tc_tiling_on_sc=False),
        out_type=jax.ShapeDtypeStruct((N_PAD,), jnp.float32),
        scratch_types=[
            pltpu.VMEM((CPWMAX * C,), jnp.int32),
            pltpu.VMEM((CPWMAX * C,), jnp.int32),
            pltpu.VMEM((NBUF, C, D // 2), jnp.int32),
            pltpu.VMEM((NBUF, C, D // 2), jnp.int32),
            pltpu.VMEM((CPWMAX * C,), jnp.float32),
            pltpu.SemaphoreType.DMA,
            pltpu.SemaphoreType.DMA,
            pltpu.SemaphoreType.DMA,
            pltpu.SemaphoreType.DMA,
            pltpu.SemaphoreType.DMA,
        ],
    )
    def k(xu_hbm, xj_hbm, is_hbm, id_hbm, out_hbm,
          idx_sv, idx_dv, rows_s, rows_d, out_v, g0, g1, g2, g3, si):
        cid = lax.axis_index("c")
        sid = lax.axis_index("s")
        wbase = jnp.where(cid == 0, sid * (CPW0 * C),
                          NS * (CPW0 * C) + sid * (CPW1 * C))
        cpw = jnp.where(cid == 0, CPW0, CPW1)
        lane = lax.iota(jnp.int32, L)
        gsem = (g0, g1, g2, g3)

        ci1 = pltpu.async_copy(is_hbm.at[pl.ds(wbase, CPWMIN * C)],
                               idx_sv.at[pl.ds(0, CPWMIN * C)], si)
        ci2 = pltpu.async_copy(id_hbm.at[pl.ds(wbase, CPWMIN * C)],
                               idx_dv.at[pl.ds(0, CPWMIN * C)], si)
        ci1.wait()
        ci2.wait()

        @pl.when(cpw > CPWMIN)
        def _idx_tail():
            rem = (CPWMAX - CPWMIN) * C
            c1 = pltpu.async_copy(is_hbm.at[pl.ds(wbase + CPWMIN * C, rem)],
                                  idx_sv.at[pl.ds(CPWMIN * C, rem)], si)
            c2 = pltpu.async_copy(id_hbm.at[pl.ds(wbase + CPWMIN * C, rem)],
                                  idx_dv.at[pl.ds(CPWMIN * C, rem)], si)
            c1.wait()
            c2.wait()

        def issue(c, b):
            pltpu.async_copy(
                xu_hbm.at[idx_sv.at[pl.ds(c * C, C)]], rows_s.at[b], gsem[b])
            pltpu.async_copy(
                xj_hbm.at[idx_dv.at[pl.ds(c * C, C)]], rows_d.at[b], gsem[b])

        def wait(b):
            pltpu.make_async_copy(
                xu_hbm.at[idx_sv.at[pl.ds(0, C)]], rows_s.at[b], gsem[b]).wait()
            pltpu.make_async_copy(
                xj_hbm.at[idx_dv.at[pl.ds(0, C)]], rows_d.at[b], gsem[b]).wait()

        fifteen = jnp.full((L,), L - 1, jnp.int32)
        unpack = functools.partial(
            plsc.unpack, format=plsc.PackFormat.INTERLEAVED)

        def compute(c, b):
            rs_ref = rows_s.at[b]
            rd_ref = rows_d.at[b]

            def group_body(g, _):
                gbase = g * L
                z = jnp.zeros((L,), jnp.float32)
                dotv, nsv, ndv = z, z, z
                for e in range(L):
                    row = gbase + e
                    acc_d, acc_s, acc_t = z, z, z
                    for j in range(D // (2 * L)):
                        xs = plsc.bitcast(
                            rs_ref[row, pl.ds(j * L, L)], jnp.bfloat16)
                        xd = plsc.bitcast(
                            rd_ref[row, pl.ds(j * L, L)], jnp.bfloat16)
                        sa, sb = unpack(xs)
                        da, db = unpack(xd)
                        acc_d = acc_d + sa * da + sb * db
                        acc_s = acc_s + sa * sa + sb * sb
                        acc_t = acc_t + da * da + db * db
                    td = plsc.cumsum(acc_d).at[fifteen].get(
                        mode="promise_in_bounds")
                    ts = plsc.cumsum(acc_s).at[fifteen].get(
                        mode="promise_in_bounds")
                    tt = plsc.cumsum(acc_t).at[fifteen].get(
                        mode="promise_in_bounds")
                    m = lane == e
                    dotv = jnp.where(m, td, dotv)
                    nsv = jnp.where(m, ts, nsv)
                    ndv = jnp.where(m, tt, ndv)
                rs = _rsqrt_nr(jnp.maximum(nsv, EPS2))
                rd = _rsqrt_nr(jnp.maximum(ndv, EPS2))
                out_v[pl.ds(c * C + gbase, L)] = dotv * rs * rd
                return _

            lax.fori_loop(0, C // L, group_body, None)

        for b in range(NBUF - 1):
            issue(b, b)

        def quad_body(cc, _):
            for b in range(NBUF):
                c = cc * NBUF + b
                nxt = c + NBUF - 1

                @pl.when(nxt < cpw)
                def _prefetch():
                    issue(nxt, (b + NBUF - 1) % NBUF)

                wait(b)
                compute(c, b)
            return _

        lax.fori_loop(0, cpw // NBUF, quad_body, None)
        pltpu.sync_copy(out_v.at[pl.ds(0, CPWMIN * C)],
                        out_hbm.at[pl.ds(wbase, CPWMIN * C)])

        @pl.when(cpw > CPWMIN)
        def _out_tail():
            rem = (CPWMAX - CPWMIN) * C
            pltpu.sync_copy(
                out_v.at[pl.ds(CPWMIN * C, rem)],
                out_hbm.at[pl.ds(wbase + CPWMIN * C, rem)])

    return k(x_user, x_job, idx_s, idx_d)


def _pack_rows(x):
    u16 = lax.bitcast_convert_type(x.astype(jnp.bfloat16), jnp.uint16)
    lo = u16[:, :D // 2].astype(jnp.uint32)
    hi = u16[:, D // 2:].astype(jnp.uint32)
    return lax.bitcast_convert_type(lo | (hi << 16), jnp.int32)


def kernel(x_user, x_job, edge_label_index):
    idx = edge_label_index.astype(jnp.int32)
    pad = N_PAD - N_EDGES
    idx_s = jnp.pad(idx[0], (0, pad))
    idx_d = jnp.pad(idx[1], (0, pad))
    out = _sc_cosine(_pack_rows(x_user), _pack_rows(x_job), idx_s, idx_d)
    return out[:N_EDGES]
